# baseline (speedup 1.0000x reference)
import jax
import jax.numpy as jnp
from jax import lax
from jax.experimental import pallas as pl
from jax.experimental.pallas import tpu as pltpu
from jax.experimental.pallas import tpu_sc as plsc

N_NODES = 10000
N_EDGES = 320000
D = 128

NC = 2
NS = 16
NW = NC * NS
CHUNK = 128
NB = 2
NI = 4
CPW = 80
E_PAD = NW * CPW * CHUNK
RPT = 632
N_PAD = NS * RPT


def _sc_body(x_hbm, sd_hbm, zero_hbm, out_hbm,
             idx, rows0, rows1, acc,
             semi0, semi1, semi2, semi3, semg0, semg1):
    c = lax.axis_index("c")
    s = lax.axis_index("s")
    wid = c * NS + s
    pltpu.sync_copy(zero_hbm, acc.at[pl.ds(s * RPT, RPT)])
    plsc.subcore_barrier()

    rows = (rows0, rows1)
    semi = (semi0, semi1, semi2, semi3)
    semg = (semg0, semg1)

    for sl in range(NI):
        pltpu.async_copy(sd_hbm.at[wid, sl], idx.at[sl], semi[sl])
    for b in range(NB):
        pltpu.make_async_copy(sd_hbm.at[wid, 0], idx.at[b], semi[b]).wait()
        pltpu.async_copy(x_hbm.at[idx.at[b, 0]], rows[b], semg[b])

    def outer(i, carry):
        j = i * NI
        for t in range(NI):
            b = t % NB
            sl2 = (t + NB) % NI
            pltpu.make_async_copy(
                x_hbm.at[idx.at[t, 0]], rows[b], semg[b]).wait()
            pltpu.sync_copy(rows[b], acc.at[idx.at[t, 1]], add=True)
            pltpu.async_copy(sd_hbm.at[wid, j + t + NI], idx.at[t], semi[t])
            pltpu.make_async_copy(
                sd_hbm.at[wid, 0], idx.at[sl2], semi[sl2]).wait()
            pltpu.async_copy(x_hbm.at[idx.at[sl2, 0]], rows[b], semg[b])
        return carry

    lax.fori_loop(0, CPW // NI, outer, 0)

    for b in range(NB):
        pltpu.make_async_copy(x_hbm.at[idx.at[0, 0]], rows[b], semg[b]).wait()
    for sl in (NB, NB + 1):
        pltpu.make_async_copy(sd_hbm.at[wid, 0], idx.at[sl], semi[sl]).wait()

    plsc.subcore_barrier()
    pltpu.sync_copy(acc.at[pl.ds(s * RPT, RPT)],
                    out_hbm.at[c, pl.ds(s * RPT, RPT)])


_sc_segment_sum = pl.kernel(
    _sc_body,
    out_type=jax.ShapeDtypeStruct((NC, N_PAD, D), jnp.float32),
    mesh=plsc.VectorSubcoreMesh(core_axis_name="c", subcore_axis_name="s"),
    scratch_types=[
        pltpu.VMEM((NI, 2, CHUNK), jnp.int32),
        pltpu.VMEM((CHUNK, D), jnp.float32),
        pltpu.VMEM((CHUNK, D), jnp.float32),
        pltpu.VMEM_SHARED((N_PAD, D), jnp.float32),
        pltpu.SemaphoreType.DMA,
        pltpu.SemaphoreType.DMA,
        pltpu.SemaphoreType.DMA,
        pltpu.SemaphoreType.DMA,
        pltpu.SemaphoreType.DMA,
        pltpu.SemaphoreType.DMA,
    ],
)


def _tc_body(acc_ref, w_ref, o1_ref, o2_ref):
    a = acc_ref[0] + acc_ref[1]
    y = jnp.dot(a, w_ref[...], preferred_element_type=jnp.float32)
    o1_ref[...] = y[:, :D]
    o2_ref[...] = jax.nn.sigmoid(y[:, D:])


_BLK = 2000

_tc_finish = pl.pallas_call(
    _tc_body,
    grid=(N_NODES // _BLK,),
    in_specs=[
        pl.BlockSpec((NC, _BLK, D), lambda i: (0, i, 0)),
        pl.BlockSpec((D, 2 * D), lambda i: (0, 0)),
    ],
    out_specs=[
        pl.BlockSpec((_BLK, D), lambda i: (i, 0)),
        pl.BlockSpec((_BLK, D), lambda i: (i, 0)),
    ],
    out_shape=[
        jax.ShapeDtypeStruct((N_NODES, D), jnp.float32),
        jax.ShapeDtypeStruct((N_NODES, D), jnp.float32),
    ],
)


def kernel(x, edge_index, W1, W2, W3):
    src = edge_index[0].astype(jnp.int32)
    dst = edge_index[1].astype(jnp.int32)
    pad_e = E_PAD - N_EDGES
    src_p = jnp.concatenate(
        [src, jnp.zeros((pad_e,), jnp.int32)]).reshape(NW, CPW, 1, CHUNK)
    dst_p = jnp.concatenate(
        [dst, jnp.full((pad_e,), N_NODES, jnp.int32)]).reshape(NW, CPW, 1, CHUNK)
    sd = jnp.concatenate([src_p, dst_p], axis=2)
    tail = jnp.concatenate(
        [jnp.zeros((NW, NI, 1, CHUNK), jnp.int32),
         jnp.full((NW, NI, 1, CHUNK), N_NODES, jnp.int32)], axis=2)
    sd = jnp.concatenate([sd, tail], axis=1)
    zero_blk = jnp.zeros((RPT, D), jnp.float32)
    part = _sc_segment_sum(x, sd, zero_blk)
    wcat = jnp.concatenate([W1, W2 + W3], axis=1)
    out_plain, out_merged = _tc_finish(part, wcat)
    return (out_plain, out_merged)


# speedup vs baseline: 3.0444x; 3.0444x over previous
import jax
import jax.numpy as jnp
from jax import lax
from jax.experimental import pallas as pl
from jax.experimental.pallas import tpu as pltpu
from jax.experimental.pallas import tpu_sc as plsc

N_NODES = 10000
N_EDGES = 320000
D = 128

NC = 2
NS = 16
NW = NC * NS
CHUNK = 128
HALF = 41
CPW = 2 * HALF
E_PAD = NW * CPW * CHUNK
RPT = 632
N_PAD = NS * RPT


def _sc_body(x_hbm, sd_hbm, zero_hbm, out_hbm,
             idx, rows0, rows1, acc, semg0, semg1):
    c = lax.axis_index("c")
    s = lax.axis_index("s")
    wid = c * NS + s
    pltpu.sync_copy(zero_hbm, acc.at[pl.ds(s * RPT, RPT)])
    plsc.subcore_barrier()

    rows = (rows0, rows1)
    semg = (semg0, semg1)

    def half_loop(h):
        pltpu.sync_copy(sd_hbm.at[wid, pl.ds(h * HALF, HALF)], idx)
        pltpu.async_copy(x_hbm.at[idx.at[0, 0]], rows[0], semg[0])

        def inner(i, carry):
            for r in range(2):
                l = i * 2 + r
                b = r
                pltpu.async_copy(
                    x_hbm.at[idx.at[l + 1, 0]], rows[1 - b], semg[1 - b])
                pltpu.make_async_copy(
                    x_hbm.at[idx.at[l, 0]], rows[b], semg[b]).wait()
                pltpu.sync_copy(rows[b], acc.at[idx.at[l, 1]], add=True)
            return carry

        lax.fori_loop(0, (HALF - 1) // 2, inner, 0)
        bt = (HALF - 1) % 2
        pltpu.make_async_copy(
            x_hbm.at[idx.at[HALF - 1, 0]], rows[bt], semg[bt]).wait()
        pltpu.sync_copy(rows[bt], acc.at[idx.at[HALF - 1, 1]], add=True)

    half_loop(0)
    half_loop(1)

    plsc.subcore_barrier()
    pltpu.sync_copy(acc.at[pl.ds(s * RPT, RPT)],
                    out_hbm.at[c, pl.ds(s * RPT, RPT)])


_sc_segment_sum = pl.kernel(
    _sc_body,
    out_type=jax.ShapeDtypeStruct((NC, N_PAD, D), jnp.float32),
    mesh=plsc.VectorSubcoreMesh(core_axis_name="c", subcore_axis_name="s"),
    scratch_types=[
        pltpu.VMEM((HALF, 2, CHUNK), jnp.int32),
        pltpu.VMEM((CHUNK, D), jnp.float32),
        pltpu.VMEM((CHUNK, D), jnp.float32),
        pltpu.VMEM_SHARED((N_PAD, D), jnp.float32),
        pltpu.SemaphoreType.DMA,
        pltpu.SemaphoreType.DMA,
    ],
)


def _tc_body(acc_ref, w_ref, o1_ref, o2_ref):
    a = acc_ref[0] + acc_ref[1]
    y = jnp.dot(a, w_ref[...], preferred_element_type=jnp.float32)
    o1_ref[...] = y[:, :D]
    o2_ref[...] = jax.nn.sigmoid(y[:, D:])


_BLK = 2000

_tc_finish = pl.pallas_call(
    _tc_body,
    grid=(N_NODES // _BLK,),
    in_specs=[
        pl.BlockSpec((NC, _BLK, D), lambda i: (0, i, 0)),
        pl.BlockSpec((D, 2 * D), lambda i: (0, 0)),
    ],
    out_specs=[
        pl.BlockSpec((_BLK, D), lambda i: (i, 0)),
        pl.BlockSpec((_BLK, D), lambda i: (i, 0)),
    ],
    out_shape=[
        jax.ShapeDtypeStruct((N_NODES, D), jnp.float32),
        jax.ShapeDtypeStruct((N_NODES, D), jnp.float32),
    ],
)


def kernel(x, edge_index, W1, W2, W3):
    src = edge_index[0].astype(jnp.int32)
    dst = edge_index[1].astype(jnp.int32)
    pad_e = E_PAD - N_EDGES
    pad_src = jnp.arange(pad_e, dtype=jnp.int32) % 8
    pad_dst = N_NODES + (jnp.arange(pad_e, dtype=jnp.int32) % (N_PAD - N_NODES))
    src_p = jnp.concatenate([src, pad_src]).reshape(NW, CPW, 1, CHUNK)
    dst_p = jnp.concatenate([dst, pad_dst]).reshape(NW, CPW, 1, CHUNK)
    sd = jnp.concatenate([src_p, dst_p], axis=2)
    zero_blk = jnp.zeros((RPT, D), jnp.float32)
    part = _sc_segment_sum(x, sd, zero_blk)
    wcat = jnp.concatenate([W1, W2 + W3], axis=1)
    out_plain, out_merged = _tc_finish(part, wcat)
    return (out_plain, out_merged)


# speedup vs baseline: 3.2945x; 1.0821x over previous
import jax
import jax.numpy as jnp
from jax import lax
from jax.experimental import pallas as pl
from jax.experimental.pallas import tpu as pltpu
from jax.experimental.pallas import tpu_sc as plsc

N_NODES = 10000
N_EDGES = 320000
D = 128

NC = 2
NS = 16
NW = NC * NS
CHUNK = 128
HALF = 41
CPW = 2 * HALF
E_PAD = NW * CPW * CHUNK
RPT = 632
N_PAD = NS * RPT


SUB = 64


def _gather_chunk(x_hbm, idx, rows, semg, l, b):
    for hf in range(2):
        pltpu.async_copy(
            x_hbm.at[idx.at[l, 0, pl.ds(hf * SUB, SUB)]],
            rows[b].at[pl.ds(hf * SUB, SUB)],
            semg[2 * b + hf])


def _wait_chunk(x_hbm, idx, rows, semg, l, b):
    for hf in range(2):
        pltpu.make_async_copy(
            x_hbm.at[idx.at[l, 0, pl.ds(hf * SUB, SUB)]],
            rows[b].at[pl.ds(hf * SUB, SUB)],
            semg[2 * b + hf]).wait()


def _sc_body(x_hbm, sd_hbm, zero_hbm, out_hbm,
             idx, rows0, rows1, acc, semg0, semg1, semg2, semg3):
    c = lax.axis_index("c")
    s = lax.axis_index("s")
    wid = c * NS + s
    pltpu.sync_copy(zero_hbm, acc.at[pl.ds(s * RPT, RPT)])
    plsc.subcore_barrier()

    rows = (rows0, rows1)
    semg = (semg0, semg1, semg2, semg3)

    def half_loop(h):
        pltpu.sync_copy(sd_hbm.at[wid, pl.ds(h * HALF, HALF)], idx)
        _gather_chunk(x_hbm, idx, rows, semg, 0, 0)

        def inner(i, carry):
            for r in range(2):
                l = i * 2 + r
                b = r
                _gather_chunk(x_hbm, idx, rows, semg, l + 1, 1 - b)
                _wait_chunk(x_hbm, idx, rows, semg, l, b)
                pltpu.sync_copy(rows[b], acc.at[idx.at[l, 1]], add=True)
            return carry

        lax.fori_loop(0, (HALF - 1) // 2, inner, 0)
        bt = (HALF - 1) % 2
        _wait_chunk(x_hbm, idx, rows, semg, HALF - 1, bt)
        pltpu.sync_copy(rows[bt], acc.at[idx.at[HALF - 1, 1]], add=True)

    half_loop(0)
    half_loop(1)

    plsc.subcore_barrier()
    pltpu.sync_copy(acc.at[pl.ds(s * RPT, RPT)],
                    out_hbm.at[c, pl.ds(s * RPT, RPT)])


_sc_segment_sum = pl.kernel(
    _sc_body,
    out_type=jax.ShapeDtypeStruct((NC, N_PAD, D), jnp.float32),
    mesh=plsc.VectorSubcoreMesh(core_axis_name="c", subcore_axis_name="s"),
    scratch_types=[
        pltpu.VMEM((HALF, 2, CHUNK), jnp.int32),
        pltpu.VMEM((CHUNK, D), jnp.float32),
        pltpu.VMEM((CHUNK, D), jnp.float32),
        pltpu.VMEM_SHARED((N_PAD, D), jnp.float32),
        pltpu.SemaphoreType.DMA,
        pltpu.SemaphoreType.DMA,
        pltpu.SemaphoreType.DMA,
        pltpu.SemaphoreType.DMA,
    ],
)


def _tc_body(acc_ref, w_ref, o1_ref, o2_ref):
    a = acc_ref[0] + acc_ref[1]
    y = jnp.dot(a, w_ref[...], preferred_element_type=jnp.float32)
    o1_ref[...] = y[:, :D]
    o2_ref[...] = jax.nn.sigmoid(y[:, D:])


_BLK = 2000

_tc_finish = pl.pallas_call(
    _tc_body,
    grid=(N_NODES // _BLK,),
    in_specs=[
        pl.BlockSpec((NC, _BLK, D), lambda i: (0, i, 0)),
        pl.BlockSpec((D, 2 * D), lambda i: (0, 0)),
    ],
    out_specs=[
        pl.BlockSpec((_BLK, D), lambda i: (i, 0)),
        pl.BlockSpec((_BLK, D), lambda i: (i, 0)),
    ],
    out_shape=[
        jax.ShapeDtypeStruct((N_NODES, D), jnp.float32),
        jax.ShapeDtypeStruct((N_NODES, D), jnp.float32),
    ],
)


def kernel(x, edge_index, W1, W2, W3):
    src = edge_index[0].astype(jnp.int32)
    dst = edge_index[1].astype(jnp.int32)
    pad_e = E_PAD - N_EDGES
    pad_src = jnp.arange(pad_e, dtype=jnp.int32) % 8
    pad_dst = N_NODES + (jnp.arange(pad_e, dtype=jnp.int32) % (N_PAD - N_NODES))
    src_p = jnp.concatenate([src, pad_src]).reshape(NW, CPW, 1, CHUNK)
    dst_p = jnp.concatenate([dst, pad_dst]).reshape(NW, CPW, 1, CHUNK)
    sd = jnp.concatenate([src_p, dst_p], axis=2)
    zero_blk = jnp.zeros((RPT, D), jnp.float32)
    part = _sc_segment_sum(x, sd, zero_blk)
    wcat = jnp.concatenate([W1, W2 + W3], axis=1)
    out_plain, out_merged = _tc_finish(part, wcat)
    return (out_plain, out_merged)


# speedup vs baseline: 3.3083x; 1.0042x over previous
import jax
import jax.numpy as jnp
from jax import lax
from jax.experimental import pallas as pl
from jax.experimental.pallas import tpu as pltpu
from jax.experimental.pallas import tpu_sc as plsc

N_NODES = 10000
N_EDGES = 320000
D = 128

NC = 2
NS = 16
NW = NC * NS
CHUNK = 128
HALF = 41
CPW = 2 * HALF
E_PAD = NW * CPW * CHUNK
RPT = 632
N_PAD = NS * RPT


SUB = 32
NSUB = 4


def _gather_chunk(x_hbm, idx, rows, semg, l, b):
    for hf in range(NSUB):
        pltpu.async_copy(
            x_hbm.at[idx.at[l, 0, pl.ds(hf * SUB, SUB)]],
            rows[b].at[pl.ds(hf * SUB, SUB)],
            semg[NSUB * b + hf])


def _wait_chunk(x_hbm, idx, rows, semg, l, b):
    for hf in range(NSUB):
        pltpu.make_async_copy(
            x_hbm.at[idx.at[l, 0, pl.ds(hf * SUB, SUB)]],
            rows[b].at[pl.ds(hf * SUB, SUB)],
            semg[NSUB * b + hf]).wait()


def _sc_body(x_hbm, sd_hbm, zero_hbm, out_hbm,
             idx, rows0, rows1, acc,
             semg0, semg1, semg2, semg3, semg4, semg5, semg6, semg7):
    c = lax.axis_index("c")
    s = lax.axis_index("s")
    wid = c * NS + s
    pltpu.sync_copy(zero_hbm, acc.at[pl.ds(s * RPT, RPT)])
    plsc.subcore_barrier()

    rows = (rows0, rows1)
    semg = (semg0, semg1, semg2, semg3, semg4, semg5, semg6, semg7)

    def half_loop(h):
        pltpu.sync_copy(sd_hbm.at[wid, pl.ds(h * HALF, HALF)], idx)
        _gather_chunk(x_hbm, idx, rows, semg, 0, 0)

        def inner(i, carry):
            for r in range(2):
                l = i * 2 + r
                b = r
                _gather_chunk(x_hbm, idx, rows, semg, l + 1, 1 - b)
                _wait_chunk(x_hbm, idx, rows, semg, l, b)
                pltpu.sync_copy(rows[b], acc.at[idx.at[l, 1]], add=True)
            return carry

        lax.fori_loop(0, (HALF - 1) // 2, inner, 0)
        bt = (HALF - 1) % 2
        _wait_chunk(x_hbm, idx, rows, semg, HALF - 1, bt)
        pltpu.sync_copy(rows[bt], acc.at[idx.at[HALF - 1, 1]], add=True)

    half_loop(0)
    half_loop(1)

    plsc.subcore_barrier()
    pltpu.sync_copy(acc.at[pl.ds(s * RPT, RPT)],
                    out_hbm.at[c, pl.ds(s * RPT, RPT)])


_sc_segment_sum = pl.kernel(
    _sc_body,
    out_type=jax.ShapeDtypeStruct((NC, N_PAD, D), jnp.float32),
    mesh=plsc.VectorSubcoreMesh(core_axis_name="c", subcore_axis_name="s"),
    scratch_types=[
        pltpu.VMEM((HALF, 2, CHUNK), jnp.int32),
        pltpu.VMEM((CHUNK, D), jnp.float32),
        pltpu.VMEM((CHUNK, D), jnp.float32),
        pltpu.VMEM_SHARED((N_PAD, D), jnp.float32),
    ] + [pltpu.SemaphoreType.DMA] * 8,
)


def _tc_body(acc_ref, w_ref, o1_ref, o2_ref):
    a = acc_ref[0] + acc_ref[1]
    y = jnp.dot(a, w_ref[...], preferred_element_type=jnp.float32)
    o1_ref[...] = y[:, :D]
    o2_ref[...] = jax.nn.sigmoid(y[:, D:])


_BLK = 2000

_tc_finish = pl.pallas_call(
    _tc_body,
    grid=(N_NODES // _BLK,),
    in_specs=[
        pl.BlockSpec((NC, _BLK, D), lambda i: (0, i, 0)),
        pl.BlockSpec((D, 2 * D), lambda i: (0, 0)),
    ],
    out_specs=[
        pl.BlockSpec((_BLK, D), lambda i: (i, 0)),
        pl.BlockSpec((_BLK, D), lambda i: (i, 0)),
    ],
    out_shape=[
        jax.ShapeDtypeStruct((N_NODES, D), jnp.float32),
        jax.ShapeDtypeStruct((N_NODES, D), jnp.float32),
    ],
)


def kernel(x, edge_index, W1, W2, W3):
    src = edge_index[0].astype(jnp.int32)
    dst = edge_index[1].astype(jnp.int32)
    pad_e = E_PAD - N_EDGES
    pad_src = jnp.arange(pad_e, dtype=jnp.int32) % 8
    pad_dst = N_NODES + (jnp.arange(pad_e, dtype=jnp.int32) % (N_PAD - N_NODES))
    src_p = jnp.concatenate([src, pad_src]).reshape(NW, CPW, 1, CHUNK)
    dst_p = jnp.concatenate([dst, pad_dst]).reshape(NW, CPW, 1, CHUNK)
    sd = jnp.concatenate([src_p, dst_p], axis=2)
    zero_blk = jnp.zeros((RPT, D), jnp.float32)
    part = _sc_segment_sum(x, sd, zero_blk)
    wcat = jnp.concatenate([W1, W2 + W3], axis=1)
    out_plain, out_merged = _tc_finish(part, wcat)
    return (out_plain, out_merged)


# speedup vs baseline: 3.3802x; 1.0217x over previous
import jax
import jax.numpy as jnp
from jax import lax
from jax.experimental import pallas as pl
from jax.experimental.pallas import tpu as pltpu
from jax.experimental.pallas import tpu_sc as plsc

N_NODES = 10000
N_EDGES = 320000
D = 128

NC = 2
NS = 16
NW = NC * NS
CHUNK = 128
HALF = 41
CPW = 2 * HALF
E_PAD = NW * CPW * CHUNK
RPT = 632
N_PAD = NS * RPT

SUB = 32
NSUB = 4


def _gather_chunk(x_hbm, idx, rows, semg, l, b):
    for hf in range(NSUB):
        pltpu.async_copy(
            x_hbm.at[idx.at[l, 0, pl.ds(hf * SUB, SUB)]],
            rows[b].at[pl.ds(hf * SUB, SUB)],
            semg[NSUB * b + hf])


def _wait_chunk(x_hbm, idx, rows, semg, l, b):
    for hf in range(NSUB):
        pltpu.make_async_copy(
            x_hbm.at[idx.at[l, 0, pl.ds(hf * SUB, SUB)]],
            rows[b].at[pl.ds(hf * SUB, SUB)],
            semg[NSUB * b + hf]).wait()


def _sc_body(x_hbm, sd_hbm, out_hbm,
             idx, rows0, rows1, acc,
             semg0, semg1, semg2, semg3, semg4, semg5, semg6, semg7):
    c = lax.axis_index("c")
    s = lax.axis_index("s")
    wid = c * NS + s
    rows = (rows0, rows1)
    semg = (semg0, semg1, semg2, semg3, semg4, semg5, semg6, semg7)

    zv = jnp.zeros((16,), jnp.float32)

    def zinit(i, carry):
        for k in range(8):
            rows1[i, pl.ds(k * 16, 16)] = zv
        return carry

    lax.fori_loop(0, CHUNK, zinit, 0)
    for q in range(4):
        pltpu.sync_copy(rows1, acc.at[pl.ds(s * RPT + q * CHUNK, CHUNK)])
    pltpu.sync_copy(rows1.at[pl.ds(0, RPT - 4 * CHUNK)],
                    acc.at[pl.ds(s * RPT + 4 * CHUNK, RPT - 4 * CHUNK)])

    pltpu.sync_copy(sd_hbm.at[wid, pl.ds(0, HALF)], idx)
    _gather_chunk(x_hbm, idx, rows, semg, 0, 0)
    plsc.subcore_barrier()

    def half_loop(h):
        if h > 0:
            pltpu.sync_copy(sd_hbm.at[wid, pl.ds(h * HALF, HALF)], idx)
            _gather_chunk(x_hbm, idx, rows, semg, 0, 0)

        def inner(i, carry):
            for r in range(2):
                l = i * 2 + r
                b = r
                _gather_chunk(x_hbm, idx, rows, semg, l + 1, 1 - b)
                _wait_chunk(x_hbm, idx, rows, semg, l, b)
                pltpu.sync_copy(rows[b], acc.at[idx.at[l, 1]], add=True)
            return carry

        lax.fori_loop(0, (HALF - 1) // 2, inner, 0)
        bt = (HALF - 1) % 2
        _wait_chunk(x_hbm, idx, rows, semg, HALF - 1, bt)
        pltpu.sync_copy(rows[bt], acc.at[idx.at[HALF - 1, 1]], add=True)

    half_loop(0)
    half_loop(1)

    plsc.subcore_barrier()
    pltpu.sync_copy(acc.at[pl.ds(s * RPT, RPT)],
                    out_hbm.at[c, pl.ds(s * RPT, RPT)])


_sc_segment_sum = pl.kernel(
    _sc_body,
    out_type=jax.ShapeDtypeStruct((NC, N_PAD, D), jnp.float32),
    mesh=plsc.VectorSubcoreMesh(core_axis_name="c", subcore_axis_name="s"),
    scratch_types=[
        pltpu.VMEM((HALF, 2, CHUNK), jnp.int32),
        pltpu.VMEM((CHUNK, D), jnp.float32),
        pltpu.VMEM((CHUNK, D), jnp.float32),
        pltpu.VMEM_SHARED((N_PAD, D), jnp.float32),
    ] + [pltpu.SemaphoreType.DMA] * 8,
)


def _tc_body(acc_ref, w_ref, o1_ref, o2_ref):
    a = acc_ref[0] + acc_ref[1]
    y = jnp.dot(a, w_ref[...], preferred_element_type=jnp.float32)
    o1_ref[...] = y[:, :D]
    o2_ref[...] = jax.nn.sigmoid(y[:, D:])


_BLK = 2000

_tc_finish = pl.pallas_call(
    _tc_body,
    grid=(N_NODES // _BLK,),
    in_specs=[
        pl.BlockSpec((NC, _BLK, D), lambda i: (0, i, 0)),
        pl.BlockSpec((D, 2 * D), lambda i: (0, 0)),
    ],
    out_specs=[
        pl.BlockSpec((_BLK, D), lambda i: (i, 0)),
        pl.BlockSpec((_BLK, D), lambda i: (i, 0)),
    ],
    out_shape=[
        jax.ShapeDtypeStruct((N_NODES, D), jnp.float32),
        jax.ShapeDtypeStruct((N_NODES, D), jnp.float32),
    ],
)


def kernel(x, edge_index, W1, W2, W3):
    src = edge_index[0].astype(jnp.int32)
    dst = edge_index[1].astype(jnp.int32)
    pad_e = E_PAD - N_EDGES
    pad_src = jnp.arange(pad_e, dtype=jnp.int32) % 8
    pad_dst = N_NODES + (jnp.arange(pad_e, dtype=jnp.int32) % (N_PAD - N_NODES))
    src_p = jnp.concatenate([src, pad_src]).reshape(NW, CPW, 1, CHUNK)
    dst_p = jnp.concatenate([dst, pad_dst]).reshape(NW, CPW, 1, CHUNK)
    sd = jnp.concatenate([src_p, dst_p], axis=2)
    part = _sc_segment_sum(x, sd)
    wcat = jnp.concatenate([W1, W2 + W3], axis=1)
    out_plain, out_merged = _tc_finish(part, wcat)
    return (out_plain, out_merged)
